# baseline (speedup 1.0000x reference)
import jax
import jax.numpy as jnp
from jax import lax
from jax.experimental import pallas as pl
from jax.experimental.pallas import tpu as pltpu
from jax.experimental.pallas import tpu_sc as plsc

NUM_BEAMS = 4
EOS = 2
PEN = 1.2
VOCAB = 100000
B = 128
CUR_LEN = 32
ROWS_BLK = 8
K = 2 * NUM_BEAMS

P = 1024
NCH = 98
VPAD = NCH * P
BIG = 2**31 - 1

_NC = 2
_NS = 16
_NW = _NC * _NS
_RPW = B // _NW


def _sc_penalize_body(scores_hbm, tok_hbm, out_hbm, row_v, tok_v, pad_v):
    wid = lax.axis_index("s") * _NC + lax.axis_index("c")
    neg = jnp.full((16,), -jnp.inf, dtype=jnp.float32)
    for j in range((VPAD - VOCAB) // 16):
        pad_v[pl.ds(16 * j, 16)] = neg
    for j in range(_RPW):
        r = wid * _RPW + j
        pltpu.sync_copy(scores_hbm.at[r], row_v)
        pltpu.sync_copy(tok_hbm.at[r], tok_v)
        idxs = [tok_v[pl.ds(16 * c, 16)] for c in range(CUR_LEN // 16)]
        vals = [plsc.load_gather(row_v, [ix]) for ix in idxs]
        for ix, g in zip(idxs, vals):
            p = jnp.where(g < 0.0, g * PEN, g * (1.0 / PEN))
            plsc.store_scatter(row_v, [ix], p)
        pltpu.sync_copy(row_v, out_hbm.at[r, pl.ds(0, VOCAB)])
        pltpu.sync_copy(pad_v, out_hbm.at[r, pl.ds(VOCAB, VPAD - VOCAB)])


import functools


@functools.cache
def _sc_penalize_kernel():
    return pl.kernel(
        _sc_penalize_body,
        out_type=jax.ShapeDtypeStruct((B, VPAD), jnp.float32),
        mesh=plsc.VectorSubcoreMesh(core_axis_name="c", subcore_axis_name="s"),
        scratch_types=[
            pltpu.VMEM((VOCAB,), jnp.float32),
            pltpu.VMEM((CUR_LEN,), jnp.int32),
            pltpu.VMEM((VPAD - VOCAB,), jnp.float32),
        ],
        compiler_params=pltpu.CompilerParams(
            needs_layout_passes=False, use_tc_tiling_on_sc=False
        ),
    )


def _top2_update(carry, v, colv):
    f1, c1, f2, c2 = carry
    gt1 = v > f1
    gt2 = v > f2
    f2n = jnp.where(gt1, f1, jnp.where(gt2, v, f2))
    c2n = jnp.where(gt1, c1, jnp.where(gt2, colv, c2))
    f1n = jnp.where(gt1, v, f1)
    c1n = jnp.where(gt1, colv, c1)
    return f1n, c1n, f2n, c2n


def _top8_kernel(x_ref, vals_ref, cols_ref, lse_ref):
    posc = lax.broadcasted_iota(jnp.int32, (ROWS_BLK, P), 1)
    ninf = jnp.float32(-jnp.inf)

    def _fold_init():
        return (
            jnp.full((ROWS_BLK, P), ninf, jnp.float32),
            jnp.full((ROWS_BLK, P), BIG, jnp.int32),
            jnp.full((ROWS_BLK, P), ninf, jnp.float32),
            jnp.full((ROWS_BLK, P), BIG, jnp.int32),
        )

    def fold_body(c, carry):
        v = x_ref[:, pl.ds(c, 1), :].reshape(ROWS_BLK, P)
        return _top2_update(carry, v, posc + c * P)

    f1, c1, f2, c2 = lax.fori_loop(0, NCH, fold_body, _fold_init())

    m = jnp.max(f1, axis=1, keepdims=True)

    def se_body(c, s):
        v = x_ref[:, pl.ds(c, 1), :].reshape(ROWS_BLK, P)
        return s + jnp.exp(v - m)

    s = lax.fori_loop(0, NCH, se_body, jnp.zeros((ROWS_BLK, P), jnp.float32))
    lse_ref[...] = m + jnp.log(jnp.sum(s, axis=1, keepdims=True))

    s2 = jnp.zeros((ROWS_BLK, P), jnp.int32)
    ecols = jnp.full((ROWS_BLK, K), -1, jnp.int32)
    slot = lax.broadcasted_iota(jnp.int32, (ROWS_BLK, K), 1)
    for k in range(K):
        mk = jnp.max(f1, axis=1, keepdims=True)
        ck = jnp.min(jnp.where(f1 == mk, c1, BIG), axis=1, keepdims=True)
        vals_ref[:, k] = mk[:, 0]
        cols_ref[:, k] = ck[:, 0]
        hit = c1 == ck
        need = jnp.any(hit & (s2 > 0))
        f1 = jnp.where(hit, f2, f1)
        c1 = jnp.where(hit, c2, c1)
        f2 = jnp.where(hit, ninf, f2)
        c2 = jnp.where(hit, BIG, c2)
        s2 = s2 | hit.astype(jnp.int32)
        ecols = jnp.where(slot == k, ck, ecols)

        def refold(ec):
            def body(c, carry):
                v = x_ref[:, pl.ds(c, 1), :].reshape(ROWS_BLK, P)
                colv = posc + c * P
                for j in range(K):
                    v = jnp.where(colv == ec[:, j:j + 1], ninf, v)
                return _top2_update(carry, v, colv)

            rf1, rc1, rf2, rc2 = lax.fori_loop(0, NCH, body, _fold_init())
            return rf1, rc1, rf2, rc2, jnp.zeros((ROWS_BLK, P), jnp.int32)

        def keep(ec):
            return f1, c1, f2, c2, s2

        f1, c1, f2, c2, s2 = lax.cond(need, refold, keep, ecols)


def _run_top8(pscores):
    grid = (B // ROWS_BLK,)
    return pl.pallas_call(
        _top8_kernel,
        grid=grid,
        in_specs=[
            pl.BlockSpec((ROWS_BLK, NCH, P), lambda i: (i, 0, 0)),
        ],
        out_specs=[
            pl.BlockSpec((ROWS_BLK, K), lambda i: (i, 0)),
            pl.BlockSpec((ROWS_BLK, K), lambda i: (i, 0)),
            pl.BlockSpec((ROWS_BLK, 1), lambda i: (i, 0)),
        ],
        out_shape=[
            jax.ShapeDtypeStruct((B, K), jnp.float32),
            jax.ShapeDtypeStruct((B, K), jnp.int32),
            jax.ShapeDtypeStruct((B, 1), jnp.float32),
        ],
    )(pscores)


@jax.jit
def kernel(scores, beam_scores, token_ids):
    pscores = _sc_penalize_kernel()(scores, token_ids)
    vals, cols, lse = _run_top8(pscores.reshape(B, NCH, P))

    bsz = B // NUM_BEAMS
    logp = vals - lse + beam_scores[:, None]
    cand_v = logp.reshape(bsz, NUM_BEAMS * K)
    cand_t = cols.reshape(bsz, NUM_BEAMS * K)
    beam = jnp.repeat(jnp.arange(NUM_BEAMS, dtype=jnp.int32), K)[None, :]
    cand_g = beam * VOCAB + cand_t

    order = jnp.lexsort((cand_g, -cand_v), axis=-1)[:, :K]
    top_v = jnp.take_along_axis(cand_v, order, axis=1)
    top_t = jnp.take_along_axis(cand_t, order, axis=1)
    top_b = jnp.take_along_axis(jnp.broadcast_to(beam, cand_g.shape), order, axis=1)

    not_eos = top_t != EOS
    keep = not_eos & (jnp.cumsum(not_eos.astype(jnp.int32), axis=1) <= NUM_BEAMS)
    sel = jnp.argsort(jnp.where(keep, 0, 1), axis=1, stable=True)[:, :NUM_BEAMS]
    kept_scores = jnp.take_along_axis(top_v, sel, axis=1)
    kept_tokens = jnp.take_along_axis(top_t, sel, axis=1)
    kept_beams = jnp.take_along_axis(top_b, sel, axis=1)
    return kept_scores, kept_tokens, kept_beams


# speedup vs baseline: 1.6155x; 1.6155x over previous
import jax
import jax.numpy as jnp
from jax import lax
from jax.experimental import pallas as pl
from jax.experimental.pallas import tpu as pltpu
from jax.experimental.pallas import tpu_sc as plsc

NUM_BEAMS = 4
EOS = 2
PEN = 1.2
VOCAB = 100000
B = 128
CUR_LEN = 32
ROWS_BLK = 8
K = 2 * NUM_BEAMS

P = 1024
NCH = 98
VPAD = NCH * P
BIG = 2**31 - 1

_NC = 2
_NS = 16
_NW = _NC * _NS
_RPW = B // _NW


def _sc_penalize_body(scores_hbm, tok_hbm, out_hbm, row_v, tok_v, pad_v):
    wid = lax.axis_index("s") * _NC + lax.axis_index("c")
    neg = jnp.full((16,), -jnp.inf, dtype=jnp.float32)
    for j in range((VPAD - VOCAB) // 16):
        pad_v[pl.ds(16 * j, 16)] = neg
    for j in range(_RPW):
        r = wid * _RPW + j
        pltpu.sync_copy(scores_hbm.at[r], row_v)
        pltpu.sync_copy(tok_hbm.at[r], tok_v)
        idxs = [tok_v[pl.ds(16 * c, 16)] for c in range(CUR_LEN // 16)]
        vals = [plsc.load_gather(row_v, [ix]) for ix in idxs]
        for ix, g in zip(idxs, vals):
            p = jnp.where(g < 0.0, g * PEN, g * (1.0 / PEN))
            plsc.store_scatter(row_v, [ix], p)
        pltpu.sync_copy(row_v, out_hbm.at[r, pl.ds(0, VOCAB)])
        pltpu.sync_copy(pad_v, out_hbm.at[r, pl.ds(VOCAB, VPAD - VOCAB)])


import functools


@functools.cache
def _sc_penalize_kernel():
    return pl.kernel(
        _sc_penalize_body,
        out_type=jax.ShapeDtypeStruct((B, VPAD), jnp.float32),
        mesh=plsc.VectorSubcoreMesh(core_axis_name="c", subcore_axis_name="s"),
        scratch_types=[
            pltpu.VMEM((VOCAB,), jnp.float32),
            pltpu.VMEM((CUR_LEN,), jnp.int32),
            pltpu.VMEM((VPAD - VOCAB,), jnp.float32),
        ],
        compiler_params=pltpu.CompilerParams(
            needs_layout_passes=False, use_tc_tiling_on_sc=False
        ),
    )


def _top8_kernel(x_ref, vals_ref, cols_ref, lse_ref, work_ref):
    posc = lax.broadcasted_iota(jnp.int32, (ROWS_BLK, P), 1)
    ninf = jnp.float32(-jnp.inf)

    f1 = jnp.full((ROWS_BLK, P), ninf, jnp.float32)
    f2 = jnp.full((ROWS_BLK, P), ninf, jnp.float32)
    f3 = jnp.full((ROWS_BLK, P), ninf, jnp.float32)
    c1 = jnp.full((ROWS_BLK, P), BIG, jnp.int32)
    c2 = jnp.full((ROWS_BLK, P), BIG, jnp.int32)
    for c in range(NCH):
        v = x_ref[:, c * P:(c + 1) * P]
        colv = posc + c * P
        gt1 = v > f1
        gt2 = v > f2
        gt3 = v > f3
        f3 = jnp.where(gt2, f2, jnp.where(gt3, v, f3))
        f2 = jnp.where(gt1, f1, jnp.where(gt2, v, f2))
        c2 = jnp.where(gt1, c1, jnp.where(gt2, colv, c2))
        f1 = jnp.where(gt1, v, f1)
        c1 = jnp.where(gt1, colv, c1)

    m = jnp.max(f1, axis=1, keepdims=True)
    s = jnp.zeros((ROWS_BLK, P), jnp.float32)
    for c in range(NCH):
        s = s + jnp.exp(x_ref[:, c * P:(c + 1) * P] - m)
    lse_ref[...] = m + jnp.log(jnp.sum(s, axis=1, keepdims=True))

    s2 = jnp.zeros((ROWS_BLK, P), jnp.bool_)
    slot = lax.broadcasted_iota(jnp.int32, (ROWS_BLK, K), 1)
    vacc = jnp.zeros((ROWS_BLK, K), jnp.float32)
    cacc = jnp.zeros((ROWS_BLK, K), jnp.int32)
    lostmax = jnp.full((ROWS_BLK, 1), ninf, jnp.float32)
    bad = jnp.zeros((), jnp.bool_)
    for k in range(K):
        mk = jnp.max(f1, axis=1, keepdims=True)
        bad = bad | jnp.any(mk <= lostmax)
        ck = jnp.min(jnp.where(f1 == mk, c1, BIG), axis=1, keepdims=True)
        vacc = jnp.where(slot == k, mk, vacc)
        cacc = jnp.where(slot == k, ck, cacc)
        hit = c1 == ck
        sec = hit & s2
        lostmax = jnp.maximum(
            lostmax, jnp.max(jnp.where(sec, f3, ninf), axis=1, keepdims=True))
        s2 = s2 | hit
        f1 = jnp.where(hit, f2, f1)
        c1 = jnp.where(hit, c2, c1)
        f2 = jnp.where(hit, ninf, f2)
        c2 = jnp.where(hit, BIG, c2)

    @pl.when(jnp.logical_not(bad))
    def _fast():
        vals_ref[...] = vacc
        cols_ref[...] = cacc

    @pl.when(bad)
    def _slow():
        work_ref[...] = x_ref[...]
        colw = lax.broadcasted_iota(jnp.int32, (ROWS_BLK, VPAD), 1)

        def body(k, acc):
            va, ca = acc
            w = work_ref[...]
            mk = jnp.max(w, axis=1, keepdims=True)
            ck = jnp.min(jnp.where(w == mk, colw, BIG), axis=1, keepdims=True)
            work_ref[...] = jnp.where(colw == ck, ninf, w)
            return jnp.where(slot == k, mk, va), jnp.where(slot == k, ck, ca)

        va, ca = lax.fori_loop(0, K, body, (vacc, cacc))
        vals_ref[...] = va
        cols_ref[...] = ca


def _run_top8(pscores):
    grid = (B // ROWS_BLK,)
    return pl.pallas_call(
        _top8_kernel,
        grid=grid,
        in_specs=[
            pl.BlockSpec((ROWS_BLK, VPAD), lambda i: (i, 0)),
        ],
        out_specs=[
            pl.BlockSpec((ROWS_BLK, K), lambda i: (i, 0)),
            pl.BlockSpec((ROWS_BLK, K), lambda i: (i, 0)),
            pl.BlockSpec((ROWS_BLK, 1), lambda i: (i, 0)),
        ],
        out_shape=[
            jax.ShapeDtypeStruct((B, K), jnp.float32),
            jax.ShapeDtypeStruct((B, K), jnp.int32),
            jax.ShapeDtypeStruct((B, 1), jnp.float32),
        ],
        scratch_shapes=[pltpu.VMEM((ROWS_BLK, VPAD), jnp.float32)],
    )(pscores)


@jax.jit
def kernel(scores, beam_scores, token_ids):
    pscores = _sc_penalize_kernel()(scores, token_ids)
    vals, cols, lse = _run_top8(pscores)

    bsz = B // NUM_BEAMS
    logp = vals - lse + beam_scores[:, None]
    cand_v = logp.reshape(bsz, NUM_BEAMS * K)
    cand_t = cols.reshape(bsz, NUM_BEAMS * K)
    beam = jnp.repeat(jnp.arange(NUM_BEAMS, dtype=jnp.int32), K)[None, :]
    cand_g = beam * VOCAB + cand_t

    order = jnp.lexsort((cand_g, -cand_v), axis=-1)[:, :K]
    top_v = jnp.take_along_axis(cand_v, order, axis=1)
    top_t = jnp.take_along_axis(cand_t, order, axis=1)
    top_b = jnp.take_along_axis(jnp.broadcast_to(beam, cand_g.shape), order, axis=1)

    not_eos = top_t != EOS
    keep = not_eos & (jnp.cumsum(not_eos.astype(jnp.int32), axis=1) <= NUM_BEAMS)
    sel = jnp.argsort(jnp.where(keep, 0, 1), axis=1, stable=True)[:, :NUM_BEAMS]
    kept_scores = jnp.take_along_axis(top_v, sel, axis=1)
    kept_tokens = jnp.take_along_axis(top_t, sel, axis=1)
    kept_beams = jnp.take_along_axis(top_b, sel, axis=1)
    return kept_scores, kept_tokens, kept_beams


# speedup vs baseline: 2.1247x; 1.3152x over previous
import jax
import jax.numpy as jnp
from jax import lax
from jax.experimental import pallas as pl
from jax.experimental.pallas import tpu as pltpu
from jax.experimental.pallas import tpu_sc as plsc

NUM_BEAMS = 4
EOS = 2
PEN = 1.2
VOCAB = 100000
B = 128
CUR_LEN = 32
ROWS_BLK = 8
K = 2 * NUM_BEAMS

P = 1024
NCH = 98
VPAD = NCH * P
BIG = 2**31 - 1

_NC = 2
_NS = 16
_NW = _NC * _NS
_RPW = B // _NW


def _sc_penalize_body(scores_hbm, tok_hbm, out_hbm, row_v, tok_v, pad_v):
    wid = lax.axis_index("s") * _NC + lax.axis_index("c")
    neg = jnp.full((16,), -jnp.inf, dtype=jnp.float32)
    for j in range((VPAD - VOCAB) // 16):
        pad_v[pl.ds(16 * j, 16)] = neg
    for j in range(_RPW):
        r = wid * _RPW + j
        pltpu.sync_copy(scores_hbm.at[r], row_v)
        pltpu.sync_copy(tok_hbm.at[r], tok_v)
        idxs = [tok_v[pl.ds(16 * c, 16)] for c in range(CUR_LEN // 16)]
        vals = [plsc.load_gather(row_v, [ix]) for ix in idxs]
        for ix, g in zip(idxs, vals):
            p = jnp.where(g < 0.0, g * PEN, g * (1.0 / PEN))
            plsc.store_scatter(row_v, [ix], p)
        pltpu.sync_copy(row_v, out_hbm.at[pl.ds(r * VPAD, VOCAB)])
        pltpu.sync_copy(pad_v, out_hbm.at[pl.ds(r * VPAD + VOCAB, VPAD - VOCAB)])


import functools


@functools.cache
def _sc_penalize_kernel():
    return pl.kernel(
        _sc_penalize_body,
        out_type=jax.ShapeDtypeStruct((B * VPAD,), jnp.float32),
        mesh=plsc.VectorSubcoreMesh(core_axis_name="c", subcore_axis_name="s"),
        scratch_types=[
            pltpu.VMEM((VOCAB,), jnp.float32),
            pltpu.VMEM((CUR_LEN,), jnp.int32),
            pltpu.VMEM((VPAD - VOCAB,), jnp.float32),
        ],
        compiler_params=pltpu.CompilerParams(needs_layout_passes=False),
    )


def _top8_kernel(x_ref, vals_ref, cols_ref, lse_ref, work_ref):
    posc = lax.broadcasted_iota(jnp.int32, (ROWS_BLK, P), 1)
    ninf = jnp.float32(-jnp.inf)

    Q = P // 4
    parts = []
    for q in range(4):
        f1 = jnp.full((ROWS_BLK, Q), ninf, jnp.float32)
        f2 = jnp.full((ROWS_BLK, Q), ninf, jnp.float32)
        f3 = jnp.full((ROWS_BLK, Q), ninf, jnp.float32)
        c1 = jnp.full((ROWS_BLK, Q), NCH, jnp.int32)
        c2 = jnp.full((ROWS_BLK, Q), NCH, jnp.int32)
        for c in range(NCH):
            v = x_ref[:, c * P + q * Q:c * P + (q + 1) * Q]
            gt1 = v > f1
            gt2 = v > f2
            gt3 = v > f3
            f3 = jnp.where(gt2, f2, jnp.where(gt3, v, f3))
            f2 = jnp.where(gt1, f1, jnp.where(gt2, v, f2))
            c2 = jnp.where(gt1, c1, jnp.where(gt2, c, c2))
            f1 = jnp.where(gt1, v, f1)
            c1 = jnp.where(gt1, c, c1)
        parts.append((f1, c1, f2, c2, f3))
    f1, c1, f2, c2, f3 = (jnp.concatenate(t, axis=1) for t in zip(*parts))
    c1 = c1 * P + posc
    c2 = c2 * P + posc

    m = jnp.max(f1, axis=1, keepdims=True)
    s = jnp.zeros((ROWS_BLK, P), jnp.float32)
    for c in range(NCH):
        s = s + jnp.exp(x_ref[:, c * P:(c + 1) * P] - m)
    lse_ref[...] = m + jnp.log(jnp.sum(s, axis=1, keepdims=True))

    s2 = jnp.zeros((ROWS_BLK, P), jnp.bool_)
    slot = lax.broadcasted_iota(jnp.int32, (ROWS_BLK, K), 1)
    vacc = jnp.zeros((ROWS_BLK, K), jnp.float32)
    cacc = jnp.zeros((ROWS_BLK, K), jnp.int32)
    lostmax = jnp.full((ROWS_BLK, 1), ninf, jnp.float32)
    bad = jnp.zeros((), jnp.bool_)
    for k in range(K):
        mk = jnp.max(f1, axis=1, keepdims=True)
        bad = bad | jnp.any(mk <= lostmax)
        ck = jnp.min(jnp.where(f1 == mk, c1, BIG), axis=1, keepdims=True)
        vacc = jnp.where(slot == k, mk, vacc)
        cacc = jnp.where(slot == k, ck, cacc)
        hit = c1 == ck
        sec = hit & s2
        lostmax = jnp.maximum(
            lostmax, jnp.max(jnp.where(sec, f3, ninf), axis=1, keepdims=True))
        s2 = s2 | hit
        f1 = jnp.where(hit, f2, f1)
        c1 = jnp.where(hit, c2, c1)
        f2 = jnp.where(hit, ninf, f2)
        c2 = jnp.where(hit, BIG, c2)

    @pl.when(jnp.logical_not(bad))
    def _fast():
        vals_ref[...] = vacc
        cols_ref[...] = cacc

    @pl.when(bad)
    def _slow():
        work_ref[...] = x_ref[...]
        colw = lax.broadcasted_iota(jnp.int32, (ROWS_BLK, VPAD), 1)

        def body(k, acc):
            va, ca = acc
            w = work_ref[...]
            mk = jnp.max(w, axis=1, keepdims=True)
            ck = jnp.min(jnp.where(w == mk, colw, BIG), axis=1, keepdims=True)
            work_ref[...] = jnp.where(colw == ck, ninf, w)
            return jnp.where(slot == k, mk, va), jnp.where(slot == k, ck, ca)

        va, ca = lax.fori_loop(0, K, body, (vacc, cacc))
        vals_ref[...] = va
        cols_ref[...] = ca


def _run_top8(pscores):
    grid = (B // ROWS_BLK,)
    return pl.pallas_call(
        _top8_kernel,
        grid=grid,
        in_specs=[
            pl.BlockSpec((ROWS_BLK, VPAD), lambda i: (i, 0)),
        ],
        out_specs=[
            pl.BlockSpec((ROWS_BLK, K), lambda i: (i, 0)),
            pl.BlockSpec((ROWS_BLK, K), lambda i: (i, 0)),
            pl.BlockSpec((ROWS_BLK, 1), lambda i: (i, 0)),
        ],
        out_shape=[
            jax.ShapeDtypeStruct((B, K), jnp.float32),
            jax.ShapeDtypeStruct((B, K), jnp.int32),
            jax.ShapeDtypeStruct((B, 1), jnp.float32),
        ],
        scratch_shapes=[pltpu.VMEM((ROWS_BLK, VPAD), jnp.float32)],
    )(pscores)


@jax.jit
def kernel(scores, beam_scores, token_ids):
    pscores = _sc_penalize_kernel()(scores, token_ids)
    vals, cols, lse = _run_top8(pscores.reshape(B, VPAD))

    bsz = B // NUM_BEAMS
    logp = vals - lse + beam_scores[:, None]
    cand_v = logp.reshape(bsz, NUM_BEAMS * K)
    cand_t = cols.reshape(bsz, NUM_BEAMS * K)
    beam = jnp.repeat(jnp.arange(NUM_BEAMS, dtype=jnp.int32), K)[None, :]
    cand_g = beam * VOCAB + cand_t

    order = jnp.lexsort((cand_g, -cand_v), axis=-1)[:, :K]
    top_v = jnp.take_along_axis(cand_v, order, axis=1)
    top_t = jnp.take_along_axis(cand_t, order, axis=1)
    top_b = jnp.take_along_axis(jnp.broadcast_to(beam, cand_g.shape), order, axis=1)

    not_eos = top_t != EOS
    keep = not_eos & (jnp.cumsum(not_eos.astype(jnp.int32), axis=1) <= NUM_BEAMS)
    sel = jnp.argsort(jnp.where(keep, 0, 1), axis=1, stable=True)[:, :NUM_BEAMS]
    kept_scores = jnp.take_along_axis(top_v, sel, axis=1)
    kept_tokens = jnp.take_along_axis(top_t, sel, axis=1)
    kept_beams = jnp.take_along_axis(top_b, sel, axis=1)
    return kept_scores, kept_tokens, kept_beams
